# Initial kernel scaffold; baseline (speedup 1.0000x reference)
import functools

import jax
import jax.numpy as jnp
from jax import lax
from jax.experimental import pallas as pl
from jax.experimental.pallas import tpu as pltpu
from jax.experimental.pallas import tpu_sc as plsc

UN = 50000
IN_ = 50000
DD = 32
EE = 800000
LL = 2

NC = 2
NS = 16
CH = 80
EPT = EE // NS
NCHUNK = EPT // CH
RPT = UN // NS
ZR = 125
NP = 50176
HPT = NP // NS

_mesh = plsc.VectorSubcoreMesh(core_axis_name="c", subcore_axis_name="s")


@functools.partial(
    pl.kernel,
    out_type=jax.ShapeDtypeStruct((2 * 2 * NP,), jnp.float32),
    mesh=_mesh,
    scratch_types=[
        pltpu.VMEM((CH,), jnp.int32),
        pltpu.VMEM((CH,), jnp.float32),
        pltpu.VMEM((HPT,), jnp.float32),
        pltpu.VMEM_SHARED((NP,), jnp.float32),
        pltpu.VMEM_SHARED((NP,), jnp.float32),
    ],
)
def _hist_kernel(rows1f, rows2f, out, idx_v, ones_v, zer_v, hacc0, hacc1):
    cid = lax.axis_index("c")
    sid = lax.axis_index("s")
    for i in range(CH // 16):
        ones_v[pl.ds(i * 16, 16)] = jnp.ones((16,), jnp.float32)

    def zinit(i, _):
        zer_v[pl.ds(i * 16, 16)] = jnp.zeros((16,), jnp.float32)
        return 0

    lax.fori_loop(0, HPT // 16, zinit, 0)
    pltpu.sync_copy(zer_v, hacc0.at[pl.ds(sid * HPT, HPT)])
    pltpu.sync_copy(zer_v, hacc1.at[pl.ds(sid * HPT, HPT)])
    plsc.subcore_barrier()

    for rowsf, hacc in ((rows1f, hacc0), (rows2f, hacc1)):
        def body(c, _):
            base = cid * EE + sid * EPT + c * CH
            pltpu.sync_copy(rowsf.at[pl.ds(base, CH)], idx_v)
            pltpu.sync_copy(ones_v, hacc.at[idx_v], add=True)
            return 0

        lax.fori_loop(0, NCHUNK, body, 0)

    plsc.subcore_barrier()
    for p, hacc in enumerate((hacc0, hacc1)):
        pltpu.sync_copy(
            hacc.at[pl.ds(sid * HPT, HPT)],
            out.at[pl.ds(p * 2 * NP + cid * NP + sid * HPT, HPT)],
        )


@functools.partial(
    pl.kernel,
    out_type=(
        jax.ShapeDtypeStruct((2 * UN, DD), jnp.float32),
        jax.ShapeDtypeStruct((2 * UN, DD), jnp.float32),
    ),
    mesh=_mesh,
    scratch_types=[
        pltpu.VMEM((CH,), jnp.int32),
        pltpu.VMEM((CH,), jnp.int32),
        pltpu.VMEM((CH, DD), jnp.float32),
        pltpu.VMEM((ZR, DD), jnp.float32),
        pltpu.VMEM_SHARED((UN, DD), jnp.float32),
        pltpu.SemaphoreType.DMA,
    ],
)
def _spmm_kernel(rows1f, cols1f, rows2f, cols2f, tab1, tab2,
                 out1, out2, rows_v, cols_v, gath_v, zer_v, acc, sem):
    cid = lax.axis_index("c")
    sid = lax.axis_index("s")

    def zinit(i, _):
        zer_v[pl.ds(i * 16, 16)] = jnp.zeros((16,), jnp.float32)
        return 0

    def zinit2(i, _):
        zer_v[pl.ds(i, 1), :] = jnp.zeros((1, DD), jnp.float32)
        return 0

    lax.fori_loop(0, ZR, zinit2, 0)

    for rowsf, colsf, tab, out in ((rows1f, cols1f, tab1, out1),
                                   (rows2f, cols2f, tab2, out2)):
        def zero_body(j, _):
            pltpu.sync_copy(zer_v, acc.at[pl.ds(sid * RPT + j * ZR, ZR)])
            return 0

        lax.fori_loop(0, RPT // ZR, zero_body, 0)
        plsc.subcore_barrier()

        def body(c, _):
            base = cid * EE + sid * EPT + c * CH
            pltpu.sync_copy(colsf.at[pl.ds(base, CH)], cols_v)
            pltpu.sync_copy(rowsf.at[pl.ds(base, CH)], rows_v)
            pltpu.async_copy(tab.at[cols_v], gath_v, sem).wait()
            pltpu.sync_copy(gath_v, acc.at[rows_v], add=True)
            return 0

        lax.fori_loop(0, NCHUNK, body, 0)
        plsc.subcore_barrier()
        pltpu.sync_copy(acc.at[pl.ds(sid * RPT, RPT)],
                        out.at[pl.ds(cid * UN + sid * RPT, RPT)])
        plsc.subcore_barrier()


BLK = 2000
NBLK = UN // BLK


def _dinv(deg):
    return jnp.where(deg > 0, lax.rsqrt(jnp.maximum(deg, 1e-12)), 0.0)


def _l2n(x):
    nrm = jnp.sqrt(jnp.sum(x * x, axis=-1, keepdims=True))
    return x / jnp.maximum(nrm, 1e-12)


def _prep_body(ue, ie, wu, bu, wi, bi, h, tab1, tab2, gu_o, gi_o):
    hh = h[...]
    duu = _dinv(hh[0, 0])[:, None]
    dii = _dinv(hh[0, 1])[:, None]
    dbu = _dinv(hh[1, 0])[:, None]
    dbi = _dinv(hh[1, 1])[:, None]
    gu = ue[...] * jax.nn.softmax(ue[...] @ wu[...] + bu[...], axis=1)
    gi = ie[...] * jax.nn.softmax(ie[...] @ wi[...] + bi[...], axis=1)
    tab1[0] = duu * gu
    tab1[1] = dii * gi
    tab2[0] = dbi * gi
    tab2[1] = dbu * gu
    gu_o[...] = gu
    gi_o[...] = gi


def _combine_body(last, o1, o2, h, up, ip, *outs):
    hh = h[...]
    duu = _dinv(hh[0, 0])[:, None]
    dii = _dinv(hh[0, 1])[:, None]
    dbu = _dinv(hh[1, 0])[:, None]
    dbi = _dinv(hh[1, 1])[:, None]
    ue = (duu * o1[0] + dbu * o2[0]) * 0.5
    ie = (dii * o1[1] + dbi * o2[1]) * 0.5
    ua = up[...] + _l2n(ue)
    ia = ip[...] + _l2n(ie)
    if last:
        (final,) = outs
        final[0] = ua
        final[1] = ia
    else:
        tab1, tab2, ua_o, ia_o = outs
        tab1[0] = duu * ue
        tab1[1] = dii * ie
        tab2[0] = dbi * ie
        tab2[1] = dbu * ue
        ua_o[...] = ua
        ia_o[...] = ia


_row_spec = pl.BlockSpec((BLK, DD), lambda i: (i, 0))
_stk_spec = pl.BlockSpec((2, BLK, DD), lambda i: (0, i, 0))
_w_spec = pl.BlockSpec((DD, DD), lambda i: (0, 0))
_b_spec = pl.BlockSpec((1, DD), lambda i: (0, 0))
_h_spec = pl.BlockSpec((2, 2, BLK), lambda i: (0, 0, i))

_f32 = jnp.float32


def _prep_call(ue, ie, wu, bu, wi, bi, h3):
    return pl.pallas_call(
        _prep_body,
        grid=(NBLK,),
        in_specs=[_row_spec, _row_spec, _w_spec, _b_spec, _w_spec, _b_spec,
                  _h_spec],
        out_specs=[_stk_spec, _stk_spec, _row_spec, _row_spec],
        out_shape=[
            jax.ShapeDtypeStruct((2, UN, DD), _f32),
            jax.ShapeDtypeStruct((2, UN, DD), _f32),
            jax.ShapeDtypeStruct((UN, DD), _f32),
            jax.ShapeDtypeStruct((UN, DD), _f32),
        ],
    )(ue, ie, wu, bu, wi, bi, h3)


def _combine_call(last, o1, o2, h3, up, ip):
    if last:
        out_specs = [_stk_spec]
        out_shape = [jax.ShapeDtypeStruct((2, UN, DD), _f32)]
    else:
        out_specs = [_stk_spec, _stk_spec, _row_spec, _row_spec]
        out_shape = [
            jax.ShapeDtypeStruct((2, UN, DD), _f32),
            jax.ShapeDtypeStruct((2, UN, DD), _f32),
            jax.ShapeDtypeStruct((UN, DD), _f32),
            jax.ShapeDtypeStruct((UN, DD), _f32),
        ]
    return pl.pallas_call(
        functools.partial(_combine_body, last),
        grid=(NBLK,),
        in_specs=[_stk_spec, _stk_spec, _h_spec, _row_spec, _row_spec],
        out_specs=out_specs,
        out_shape=out_shape,
    )(o1, o2, h3, up, ip)


def kernel(user_emb, item_emb, gating_weightu, gating_weightub,
           gating_weighti, gating_weightib,
           uu_rows, uu_cols, ii_rows, ii_cols, ui_rows, ui_cols):
    b_u = ui_rows[:EE]
    b_i = ui_cols[:EE] - UN

    off = jnp.int32(UN)
    rows1f = jnp.concatenate([uu_rows, ii_rows])
    cols1f = jnp.concatenate([uu_cols, ii_cols + off])
    rows2f = jnp.concatenate([b_u, b_i])
    cols2f = jnp.concatenate([b_i, b_u + off])

    hflat = _hist_kernel(rows1f, rows2f)
    h3 = hflat.reshape(2, 2, NP)

    tab1, tab2, ua, ia = _prep_call(
        user_emb, item_emb, gating_weightu, gating_weightub,
        gating_weighti, gating_weightib, h3)

    t1 = tab1.reshape(2 * UN, DD)
    t2 = tab2.reshape(2 * UN, DD)
    final = None
    for layer in range(LL):
        o1, o2 = _spmm_kernel(rows1f, cols1f, rows2f, cols2f, t1, t2)
        o1 = o1.reshape(2, UN, DD)
        o2 = o2.reshape(2, UN, DD)
        if layer + 1 < LL:
            tab1, tab2, ua, ia = _combine_call(False, o1, o2, h3, ua, ia)
            t1 = tab1.reshape(2 * UN, DD)
            t2 = tab2.reshape(2 * UN, DD)
        else:
            (final,) = _combine_call(True, o1, o2, h3, ua, ia)
    return final.reshape(2 * UN, DD)


# speedup vs baseline: 13.1834x; 13.1834x over previous
import functools

import jax
import jax.numpy as jnp
from jax import lax
from jax.experimental import pallas as pl
from jax.experimental.pallas import tpu as pltpu
from jax.experimental.pallas import tpu_sc as plsc

UN = 50000
IN_ = 50000
DD = 32
EE = 800000
LL = 2

NC = 2
NS = 16
CH = 80
EPT = EE // NS
NCHUNK = EPT // CH
UNP = 50176
RPT = UNP // NS
ZR = 112
NP = 50176
HPT = NP // NS

_mesh = plsc.VectorSubcoreMesh(core_axis_name="c", subcore_axis_name="s")


@functools.partial(
    pl.kernel,
    out_type=jax.ShapeDtypeStruct((2 * 2 * NP,), jnp.float32),
    mesh=_mesh,
    compiler_params=pltpu.CompilerParams(use_tc_tiling_on_sc=False),
    scratch_types=[
        pltpu.VMEM((CH,), jnp.int32),
        pltpu.VMEM((CH,), jnp.float32),
        pltpu.VMEM((HPT,), jnp.float32),
        pltpu.VMEM_SHARED((NP,), jnp.float32),
        pltpu.VMEM_SHARED((NP,), jnp.float32),
    ],
)
def _hist_kernel(rows1f, rows2f, out, idx_v, ones_v, zer_v, hacc0, hacc1):
    cid = lax.axis_index("c")
    sid = lax.axis_index("s")
    for i in range(CH // 16):
        ones_v[pl.ds(i * 16, 16)] = jnp.ones((16,), jnp.float32)

    def zinit(i, _):
        zer_v[pl.ds(i * 16, 16)] = jnp.zeros((16,), jnp.float32)
        return 0

    lax.fori_loop(0, HPT // 16, zinit, 0)
    pltpu.sync_copy(zer_v, hacc0.at[pl.ds(sid * HPT, HPT)])
    pltpu.sync_copy(zer_v, hacc1.at[pl.ds(sid * HPT, HPT)])
    plsc.subcore_barrier()

    for rowsf, hacc in ((rows1f, hacc0), (rows2f, hacc1)):
        def body(c, _):
            base = cid * EE + sid * EPT + c * CH
            pltpu.sync_copy(rowsf.at[pl.ds(base, CH)], idx_v)
            pltpu.sync_copy(ones_v, hacc.at[idx_v], add=True)
            return 0

        lax.fori_loop(0, NCHUNK, body, 0)

    plsc.subcore_barrier()
    for p, hacc in enumerate((hacc0, hacc1)):
        pltpu.sync_copy(hacc.at[pl.ds(sid * HPT, HPT)], zer_v)
        pltpu.sync_copy(
            zer_v,
            out.at[pl.ds(p * 2 * NP + cid * NP + sid * HPT, HPT)],
        )


@functools.partial(
    pl.kernel,
    out_type=(
        jax.ShapeDtypeStruct((2 * UNP, DD), jnp.float32),
        jax.ShapeDtypeStruct((2 * UNP, DD), jnp.float32),
    ),
    mesh=_mesh,
    compiler_params=pltpu.CompilerParams(use_tc_tiling_on_sc=False),
    scratch_types=[
        pltpu.VMEM((CH,), jnp.int32),
        pltpu.VMEM((CH,), jnp.int32),
        pltpu.VMEM((CH, DD), jnp.float32),
        pltpu.VMEM((ZR, DD), jnp.float32),
        pltpu.VMEM((ZR, DD), jnp.float32),
        pltpu.VMEM_SHARED((UNP, DD), jnp.float32),
        pltpu.SemaphoreType.DMA,
    ],
)
def _spmm_kernel(rows1f, cols1f, rows2f, cols2f, tab1, tab2,
                 out1, out2, rows_v, cols_v, gath_v, zer_v, stage_v, acc, sem):
    cid = lax.axis_index("c")
    sid = lax.axis_index("s")

    z16 = jnp.zeros((16,), jnp.float32)
    for r in range(ZR):
        zer_v[r, pl.ds(0, 16)] = z16
        zer_v[r, pl.ds(16, 16)] = z16

    for rowsf, colsf, tab, out in ((rows1f, cols1f, tab1, out1),
                                   (rows2f, cols2f, tab2, out2)):
        def zero_body(j, _):
            pltpu.sync_copy(zer_v, acc.at[pl.ds(sid * RPT + j * ZR, ZR)])
            return 0

        lax.fori_loop(0, RPT // ZR, zero_body, 0)
        plsc.subcore_barrier()

        def body(c, _):
            base = cid * EE + sid * EPT + c * CH
            pltpu.sync_copy(colsf.at[pl.ds(base, CH)], cols_v)
            pltpu.sync_copy(rowsf.at[pl.ds(base, CH)], rows_v)
            pltpu.async_copy(tab.at[cols_v], gath_v, sem).wait()
            pltpu.sync_copy(gath_v, acc.at[rows_v], add=True)
            return 0

        lax.fori_loop(0, NCHUNK, body, 0)
        plsc.subcore_barrier()

        def out_body(j, _):
            pltpu.sync_copy(acc.at[pl.ds(sid * RPT + j * ZR, ZR)], stage_v)
            pltpu.sync_copy(stage_v,
                            out.at[pl.ds(cid * UNP + sid * RPT + j * ZR, ZR)])
            return 0

        lax.fori_loop(0, RPT // ZR, out_body, 0)
        plsc.subcore_barrier()


BLK = 2000
NBLK = UN // BLK


def _dinv(deg):
    return jnp.where(deg > 0, lax.rsqrt(jnp.maximum(deg, 1e-12)), 0.0)


def _l2n(x):
    nrm = jnp.sqrt(jnp.sum(x * x, axis=-1, keepdims=True))
    return x / jnp.maximum(nrm, 1e-12)


def _prep_body(ue, ie, wu, bu, wi, bi, huu, hii, hbu, hbi,
               tab1, tab2, gu_o, gi_o):
    duu = _dinv(huu[...])
    dii = _dinv(hii[...])
    dbu = _dinv(hbu[...])
    dbi = _dinv(hbi[...])
    gu = ue[...] * jax.nn.softmax(ue[...] @ wu[...] + bu[...], axis=1)
    gi = ie[...] * jax.nn.softmax(ie[...] @ wi[...] + bi[...], axis=1)
    tab1[0] = duu * gu
    tab1[1] = dii * gi
    tab2[0] = dbi * gi
    tab2[1] = dbu * gu
    gu_o[...] = gu
    gi_o[...] = gi


def _combine_body(last, o1, o2, huu, hii, hbu, hbi, up, ip, *outs):
    duu = _dinv(huu[...])
    dii = _dinv(hii[...])
    dbu = _dinv(hbu[...])
    dbi = _dinv(hbi[...])
    ue = (duu * o1[0] + dbu * o2[0]) * 0.5
    ie = (dii * o1[1] + dbi * o2[1]) * 0.5
    ua = up[...] + _l2n(ue)
    ia = ip[...] + _l2n(ie)
    if last:
        (final,) = outs
        final[0] = ua
        final[1] = ia
    else:
        tab1, tab2, ua_o, ia_o = outs
        tab1[0] = duu * ue
        tab1[1] = dii * ie
        tab2[0] = dbi * ie
        tab2[1] = dbu * ue
        ua_o[...] = ua
        ia_o[...] = ia


_row_spec = pl.BlockSpec((BLK, DD), lambda i: (i, 0))
_stk_spec = pl.BlockSpec((2, BLK, DD), lambda i: (0, i, 0))
_w_spec = pl.BlockSpec((DD, DD), lambda i: (0, 0))
_b_spec = pl.BlockSpec((1, DD), lambda i: (0, 0))
_c_spec = pl.BlockSpec((BLK, 1), lambda i: (i, 0))

_f32 = jnp.float32


def _prep_call(ue, ie, wu, bu, wi, bi, hs):
    return pl.pallas_call(
        _prep_body,
        grid=(NBLK,),
        in_specs=[_row_spec, _row_spec, _w_spec, _b_spec, _w_spec, _b_spec,
                  _c_spec, _c_spec, _c_spec, _c_spec],
        out_specs=[_stk_spec, _stk_spec, _row_spec, _row_spec],
        out_shape=[
            jax.ShapeDtypeStruct((2, UN, DD), _f32),
            jax.ShapeDtypeStruct((2, UN, DD), _f32),
            jax.ShapeDtypeStruct((UN, DD), _f32),
            jax.ShapeDtypeStruct((UN, DD), _f32),
        ],
    )(ue, ie, wu, bu, wi, bi, *hs)


def _combine_call(last, o1, o2, hs, up, ip):
    if last:
        out_specs = [_stk_spec]
        out_shape = [jax.ShapeDtypeStruct((2, UN, DD), _f32)]
    else:
        out_specs = [_stk_spec, _stk_spec, _row_spec, _row_spec]
        out_shape = [
            jax.ShapeDtypeStruct((2, UN, DD), _f32),
            jax.ShapeDtypeStruct((2, UN, DD), _f32),
            jax.ShapeDtypeStruct((UN, DD), _f32),
            jax.ShapeDtypeStruct((UN, DD), _f32),
        ]
    return pl.pallas_call(
        functools.partial(_combine_body, last),
        grid=(NBLK,),
        in_specs=[_stk_spec, _stk_spec, _c_spec, _c_spec, _c_spec, _c_spec,
                  _row_spec, _row_spec],
        out_specs=out_specs,
        out_shape=out_shape,
    )(o1, o2, *hs, up, ip)


def kernel(user_emb, item_emb, gating_weightu, gating_weightub,
           gating_weighti, gating_weightib,
           uu_rows, uu_cols, ii_rows, ii_cols, ui_rows, ui_cols):
    b_u = ui_rows[:EE]
    b_i = ui_cols[:EE] - UN

    off = jnp.int32(UN)
    rows1f = jnp.concatenate([uu_rows, ii_rows])
    cols1f = jnp.concatenate([uu_cols, ii_cols + off])
    rows2f = jnp.concatenate([b_u, b_i])
    cols2f = jnp.concatenate([b_i, b_u + off])

    hflat = _hist_kernel(rows1f, rows2f)
    h4 = hflat.reshape(4, NP)
    hs = tuple(h4[k].reshape(NP, 1) for k in range(4))

    tab1, tab2, ua, ia = _prep_call(
        user_emb, item_emb, gating_weightu, gating_weightub,
        gating_weighti, gating_weightib, hs)

    t1 = tab1.reshape(2 * UN, DD)
    t2 = tab2.reshape(2 * UN, DD)
    final = None
    for layer in range(LL):
        o1, o2 = _spmm_kernel(rows1f, cols1f, rows2f, cols2f, t1, t2)
        o1 = o1.reshape(2, UNP, DD)
        o2 = o2.reshape(2, UNP, DD)
        if layer + 1 < LL:
            tab1, tab2, ua, ia = _combine_call(False, o1, o2, hs, ua, ia)
            t1 = tab1.reshape(2 * UN, DD)
            t2 = tab2.reshape(2 * UN, DD)
        else:
            (final,) = _combine_call(True, o1, o2, hs, ua, ia)
    return final.reshape(2 * UN, DD)


# speedup vs baseline: 32.4559x; 2.4619x over previous
import functools

import jax
import jax.numpy as jnp
from jax import lax
from jax.experimental import pallas as pl
from jax.experimental.pallas import tpu as pltpu
from jax.experimental.pallas import tpu_sc as plsc

UN = 50000
IN_ = 50000
DD = 32
EE = 800000
LL = 2

NC = 2
NS = 16
CH = 400
EPT = EE // NS
NCHUNK = EPT // CH
UNP = 50176
RPT = UNP // NS
ZR = 112
NP = 50176
HPT = NP // NS

_mesh = plsc.VectorSubcoreMesh(core_axis_name="c", subcore_axis_name="s")


@functools.partial(
    pl.kernel,
    out_type=jax.ShapeDtypeStruct((2 * 2 * NP,), jnp.float32),
    mesh=_mesh,
    compiler_params=pltpu.CompilerParams(use_tc_tiling_on_sc=False),
    scratch_types=[
        pltpu.VMEM((CH,), jnp.int32),
        pltpu.VMEM((CH,), jnp.float32),
        pltpu.VMEM((HPT,), jnp.float32),
        pltpu.VMEM_SHARED((NP,), jnp.float32),
        pltpu.VMEM_SHARED((NP,), jnp.float32),
    ],
)
def _hist_kernel(rows1f, rows2f, out, idx_v, ones_v, zer_v, hacc0, hacc1):
    cid = lax.axis_index("c")
    sid = lax.axis_index("s")
    for i in range(CH // 16):
        ones_v[pl.ds(i * 16, 16)] = jnp.ones((16,), jnp.float32)

    def zinit(i, _):
        zer_v[pl.ds(i * 16, 16)] = jnp.zeros((16,), jnp.float32)
        return 0

    lax.fori_loop(0, HPT // 16, zinit, 0)
    pltpu.sync_copy(zer_v, hacc0.at[pl.ds(sid * HPT, HPT)])
    pltpu.sync_copy(zer_v, hacc1.at[pl.ds(sid * HPT, HPT)])
    plsc.subcore_barrier()

    for rowsf, hacc in ((rows1f, hacc0), (rows2f, hacc1)):
        def body(c, _):
            base = cid * EE + sid * EPT + c * CH
            pltpu.sync_copy(rowsf.at[pl.ds(base, CH)], idx_v)
            pltpu.sync_copy(ones_v, hacc.at[idx_v], add=True)
            return 0

        lax.fori_loop(0, NCHUNK, body, 0)

    plsc.subcore_barrier()
    for p, hacc in enumerate((hacc0, hacc1)):
        pltpu.sync_copy(hacc.at[pl.ds(sid * HPT, HPT)], zer_v)
        pltpu.sync_copy(
            zer_v,
            out.at[pl.ds(p * 2 * NP + cid * NP + sid * HPT, HPT)],
        )


@functools.partial(
    pl.kernel,
    out_type=(
        jax.ShapeDtypeStruct((2 * UNP, DD), jnp.float32),
        jax.ShapeDtypeStruct((2 * UNP, DD), jnp.float32),
    ),
    mesh=_mesh,
    compiler_params=pltpu.CompilerParams(use_tc_tiling_on_sc=False),
    scratch_types=[
        pltpu.VMEM((CH,), jnp.int32),
        pltpu.VMEM((CH,), jnp.int32),
        pltpu.VMEM((CH, DD), jnp.float32),
        pltpu.VMEM((ZR, DD), jnp.float32),
        pltpu.VMEM((ZR, DD), jnp.float32),
        pltpu.VMEM_SHARED((UNP, DD), jnp.float32),
        pltpu.SemaphoreType.DMA,
    ],
)
def _spmm_kernel(rows1f, cols1f, rows2f, cols2f, tab1, tab2,
                 out1, out2, rows_v, cols_v, gath_v, zer_v, stage_v, acc, sem):
    cid = lax.axis_index("c")
    sid = lax.axis_index("s")

    z16 = jnp.zeros((16,), jnp.float32)
    for r in range(ZR):
        zer_v[r, pl.ds(0, 16)] = z16
        zer_v[r, pl.ds(16, 16)] = z16

    for rowsf, colsf, tab, out in ((rows1f, cols1f, tab1, out1),
                                   (rows2f, cols2f, tab2, out2)):
        def zero_body(j, _):
            pltpu.sync_copy(zer_v, acc.at[pl.ds(sid * RPT + j * ZR, ZR)])
            return 0

        lax.fori_loop(0, RPT // ZR, zero_body, 0)
        plsc.subcore_barrier()

        def body(c, _):
            base = cid * EE + sid * EPT + c * CH
            pltpu.sync_copy(colsf.at[pl.ds(base, CH)], cols_v)
            pltpu.sync_copy(rowsf.at[pl.ds(base, CH)], rows_v)
            pltpu.async_copy(tab.at[cols_v], gath_v, sem).wait()
            pltpu.sync_copy(gath_v, acc.at[rows_v], add=True)
            return 0

        lax.fori_loop(0, NCHUNK, body, 0)
        plsc.subcore_barrier()

        def out_body(j, _):
            pltpu.sync_copy(acc.at[pl.ds(sid * RPT + j * ZR, ZR)], stage_v)
            pltpu.sync_copy(stage_v,
                            out.at[pl.ds(cid * UNP + sid * RPT + j * ZR, ZR)])
            return 0

        lax.fori_loop(0, RPT // ZR, out_body, 0)
        plsc.subcore_barrier()


BLK = 2000
NBLK = UN // BLK


def _dinv(deg):
    return jnp.where(deg > 0, lax.rsqrt(jnp.maximum(deg, 1e-12)), 0.0)


def _l2n(x):
    nrm = jnp.sqrt(jnp.sum(x * x, axis=-1, keepdims=True))
    return x / jnp.maximum(nrm, 1e-12)


def _prep_body(ue, ie, wu, bu, wi, bi, huu, hii, hbu, hbi,
               tab1, tab2, gu_o, gi_o):
    duu = _dinv(huu[...])
    dii = _dinv(hii[...])
    dbu = _dinv(hbu[...])
    dbi = _dinv(hbi[...])
    gu = ue[...] * jax.nn.softmax(ue[...] @ wu[...] + bu[...], axis=1)
    gi = ie[...] * jax.nn.softmax(ie[...] @ wi[...] + bi[...], axis=1)
    tab1[0] = duu * gu
    tab1[1] = dii * gi
    tab2[0] = dbi * gi
    tab2[1] = dbu * gu
    gu_o[...] = gu
    gi_o[...] = gi


def _combine_body(last, o1, o2, huu, hii, hbu, hbi, up, ip, *outs):
    duu = _dinv(huu[...])
    dii = _dinv(hii[...])
    dbu = _dinv(hbu[...])
    dbi = _dinv(hbi[...])
    ue = (duu * o1[0] + dbu * o2[0]) * 0.5
    ie = (dii * o1[1] + dbi * o2[1]) * 0.5
    ua = up[...] + _l2n(ue)
    ia = ip[...] + _l2n(ie)
    if last:
        (final,) = outs
        final[0] = ua
        final[1] = ia
    else:
        tab1, tab2, ua_o, ia_o = outs
        tab1[0] = duu * ue
        tab1[1] = dii * ie
        tab2[0] = dbi * ie
        tab2[1] = dbu * ue
        ua_o[...] = ua
        ia_o[...] = ia


_row_spec = pl.BlockSpec((BLK, DD), lambda i: (i, 0))
_stk_spec = pl.BlockSpec((2, BLK, DD), lambda i: (0, i, 0))
_w_spec = pl.BlockSpec((DD, DD), lambda i: (0, 0))
_b_spec = pl.BlockSpec((1, DD), lambda i: (0, 0))
_c_spec = pl.BlockSpec((BLK, 1), lambda i: (i, 0))

_f32 = jnp.float32


def _prep_call(ue, ie, wu, bu, wi, bi, hs):
    return pl.pallas_call(
        _prep_body,
        grid=(NBLK,),
        in_specs=[_row_spec, _row_spec, _w_spec, _b_spec, _w_spec, _b_spec,
                  _c_spec, _c_spec, _c_spec, _c_spec],
        out_specs=[_stk_spec, _stk_spec, _row_spec, _row_spec],
        out_shape=[
            jax.ShapeDtypeStruct((2, UN, DD), _f32),
            jax.ShapeDtypeStruct((2, UN, DD), _f32),
            jax.ShapeDtypeStruct((UN, DD), _f32),
            jax.ShapeDtypeStruct((UN, DD), _f32),
        ],
    )(ue, ie, wu, bu, wi, bi, *hs)


def _combine_call(last, o1, o2, hs, up, ip):
    if last:
        out_specs = [_stk_spec]
        out_shape = [jax.ShapeDtypeStruct((2, UN, DD), _f32)]
    else:
        out_specs = [_stk_spec, _stk_spec, _row_spec, _row_spec]
        out_shape = [
            jax.ShapeDtypeStruct((2, UN, DD), _f32),
            jax.ShapeDtypeStruct((2, UN, DD), _f32),
            jax.ShapeDtypeStruct((UN, DD), _f32),
            jax.ShapeDtypeStruct((UN, DD), _f32),
        ]
    return pl.pallas_call(
        functools.partial(_combine_body, last),
        grid=(NBLK,),
        in_specs=[_stk_spec, _stk_spec, _c_spec, _c_spec, _c_spec, _c_spec,
                  _row_spec, _row_spec],
        out_specs=out_specs,
        out_shape=out_shape,
    )(o1, o2, *hs, up, ip)


def kernel(user_emb, item_emb, gating_weightu, gating_weightub,
           gating_weighti, gating_weightib,
           uu_rows, uu_cols, ii_rows, ii_cols, ui_rows, ui_cols):
    b_u = ui_rows[:EE]
    b_i = ui_cols[:EE] - UN

    off = jnp.int32(UN)
    rows1f = jnp.concatenate([uu_rows, ii_rows])
    cols1f = jnp.concatenate([uu_cols, ii_cols + off])
    rows2f = jnp.concatenate([b_u, b_i])
    cols2f = jnp.concatenate([b_i, b_u + off])

    hflat = _hist_kernel(rows1f, rows2f)
    h4 = hflat.reshape(4, NP)
    hs = tuple(h4[k].reshape(NP, 1) for k in range(4))

    tab1, tab2, ua, ia = _prep_call(
        user_emb, item_emb, gating_weightu, gating_weightub,
        gating_weighti, gating_weightib, hs)

    t1 = tab1.reshape(2 * UN, DD)
    t2 = tab2.reshape(2 * UN, DD)
    final = None
    for layer in range(LL):
        o1, o2 = _spmm_kernel(rows1f, cols1f, rows2f, cols2f, t1, t2)
        o1 = o1.reshape(2, UNP, DD)
        o2 = o2.reshape(2, UNP, DD)
        if layer + 1 < LL:
            tab1, tab2, ua, ia = _combine_call(False, o1, o2, hs, ua, ia)
            t1 = tab1.reshape(2 * UN, DD)
            t2 = tab2.reshape(2 * UN, DD)
        else:
            (final,) = _combine_call(True, o1, o2, hs, ua, ia)
    return final.reshape(2 * UN, DD)
